# baseline (speedup 1.0000x reference)
import jax
import jax.numpy as jnp
from jax.experimental import pallas as pl
from jax.experimental.pallas import tpu as pltpu

CHUNKS = (512, 1024, 2048, 4096, 4096, 4096, 4096, 4096, 4096, 4096, 512)
MAX_CHUNK = max(CHUNKS)
N_BUF = 3


def _gating_body(x_hbm, w1_ref, b1_ref, w2_ref, b2_ref, out_ref, out2_ref,
                 buf0, buf1, buf2, sem0a, sem0b, sem1a, sem1b, sem2a, sem2b):
    bufs = (buf0, buf1, buf2)
    sems = ((sem0a, sem0b), (sem1a, sem1b), (sem2a, sem2b))
    w1b = w1_ref[...].astype(jnp.bfloat16)
    w2b = w2_ref[...].astype(jnp.bfloat16)
    b1 = b1_ref[...]
    b2 = b2_ref[...]

    offs = []
    o = 0
    for c in CHUNKS:
        offs.append(o)
        o += c

    def copies_in(i):
        c = CHUNKS[i]
        half = c // 2
        b = bufs[i % N_BUF]
        sa, sb = sems[i % N_BUF]
        return (
            pltpu.make_async_copy(
                x_hbm.at[pl.ds(offs[i], half), :],
                b.at[pl.ds(0, half), :], sa),
            pltpu.make_async_copy(
                x_hbm.at[pl.ds(offs[i] + half, half), :],
                b.at[pl.ds(half, half), :], sb),
        )

    def start(i):
        for cp in copies_in(i):
            cp.start()

    def wait(i):
        for cp in copies_in(i):
            cp.wait()

    start(0)
    start(1)
    for i, c in enumerate(CHUNKS):
        if i + 2 < len(CHUNKS):
            start(i + 2)
        wait(i)
        xb = bufs[i % N_BUF][pl.ds(0, c), :].astype(jnp.bfloat16)
        h_t = jnp.tanh(
            jax.lax.dot_general(w1b, xb, (((1,), (1,)), ((), ())),
                                preferred_element_type=jnp.float32)
            + b1
        )
        gates_t = (
            jax.lax.dot_general(w2b, h_t.astype(jnp.bfloat16),
                                (((1,), (0,)), ((), ())),
                                preferred_element_type=jnp.float32)
            + b2
        )
        out_ref[:, pl.ds(offs[i], c)] = gates_t
        out2_ref[:, pl.ds(offs[i], c)] = gates_t


@jax.jit
def _gating(x, w1, b1, w2, b2):
    tokens, feats = x.shape
    num_experts = w1.shape[0]
    gates_t = pl.pallas_call(
        _gating_body,
        in_specs=[
            pl.BlockSpec(memory_space=pltpu.MemorySpace.HBM),
            pl.BlockSpec((num_experts, feats), lambda: (0, 0)),
            pl.BlockSpec((num_experts, 1), lambda: (0, 0)),
            pl.BlockSpec((num_experts, num_experts), lambda: (0, 0)),
            pl.BlockSpec((num_experts, 1), lambda: (0, 0)),
        ],
        out_specs=[
            pl.BlockSpec((num_experts, tokens), lambda: (0, 0)),
            pl.BlockSpec((num_experts, tokens), lambda: (0, 0)),
        ],
        out_shape=[
            jax.ShapeDtypeStruct((num_experts, tokens), jnp.float32),
            jax.ShapeDtypeStruct((num_experts, tokens), jnp.float32),
        ],
        scratch_shapes=[
            pltpu.VMEM((MAX_CHUNK, 768), jnp.float32),
            pltpu.VMEM((MAX_CHUNK, 768), jnp.float32),
            pltpu.VMEM((MAX_CHUNK, 768), jnp.float32),
            pltpu.SemaphoreType.DMA,
            pltpu.SemaphoreType.DMA,
            pltpu.SemaphoreType.DMA,
            pltpu.SemaphoreType.DMA,
            pltpu.SemaphoreType.DMA,
            pltpu.SemaphoreType.DMA,
        ],
    )(x, w1, b1, w2, b2)
    return gates_t[0].T, gates_t[1].T


def kernel(x, W1, b1, W2, b2, train):
    out, gates = _gating(x, W1, b1.reshape(-1, 1), W2, b2.reshape(-1, 1))
    return (out, gates)


# speedup vs baseline: 1.0466x; 1.0466x over previous
import jax
import jax.numpy as jnp
from jax.experimental import pallas as pl
from jax.experimental.pallas import tpu as pltpu

TOKEN_BLOCK = 4096


def _gating_block(x_ref, w1_ref, b1_ref, w2_ref, b2_ref, out_ref, out2_ref):
    xb = x_ref[...].astype(jnp.bfloat16)
    w1b = w1_ref[...].astype(jnp.bfloat16)
    h_t = jnp.tanh(
        jax.lax.dot_general(w1b, xb, (((1,), (1,)), ((), ())),
                            preferred_element_type=jnp.float32)
        + b1_ref[...]
    )
    w2b = w2_ref[...].astype(jnp.bfloat16)
    gates_t = (
        jax.lax.dot_general(w2b, h_t.astype(jnp.bfloat16),
                            (((1,), (0,)), ((), ())),
                            preferred_element_type=jnp.float32)
        + b2_ref[...]
    )
    out_ref[...] = gates_t
    out2_ref[...] = gates_t


@jax.jit
def _gating(x, w1, b1, w2, b2):
    tokens, feats = x.shape
    num_experts = w1.shape[0]
    grid = (tokens // TOKEN_BLOCK,)
    gates_t = pl.pallas_call(
        _gating_block,
        grid=grid,
        in_specs=[
            pl.BlockSpec((TOKEN_BLOCK, feats), lambda i: (i, 0)),
            pl.BlockSpec((num_experts, feats), lambda i: (0, 0)),
            pl.BlockSpec((num_experts, 1), lambda i: (0, 0)),
            pl.BlockSpec((num_experts, num_experts), lambda i: (0, 0)),
            pl.BlockSpec((num_experts, 1), lambda i: (0, 0)),
        ],
        out_specs=[
            pl.BlockSpec((num_experts, TOKEN_BLOCK), lambda i: (0, i)),
            pl.BlockSpec((num_experts, TOKEN_BLOCK), lambda i: (0, i)),
        ],
        out_shape=[
            jax.ShapeDtypeStruct((num_experts, tokens), jnp.float32),
            jax.ShapeDtypeStruct((num_experts, tokens), jnp.float32),
        ],
        compiler_params=pltpu.CompilerParams(
            dimension_semantics=("parallel",),
        ),
    )(x, w1, b1, w2, b2)
    return gates_t[0].T, gates_t[1].T


def kernel(x, W1, b1, W2, b2, train):
    out, gates = _gating(x, W1, b1.reshape(-1, 1), W2, b2.reshape(-1, 1))
    return (out, gates)
